# baseline (speedup 1.0000x reference)
import functools

import jax
import jax.numpy as jnp
from jax import lax
from jax.experimental import pallas as pl
from jax.experimental.pallas import tpu as pltpu
from jax.experimental.pallas import tpu_sc as plsc

_L = 16
_U = 2


@functools.lru_cache(maxsize=None)
def _make_embed(D, B, S):
    info = plsc.get_sparse_core_info()
    NC, NS = info.num_cores, info.num_subcores
    NW = NC * NS
    N = B * S
    POS_PER_W = S // NW
    KP = 8
    NCH = POS_PER_W // KP
    NBUF = 5
    NIT = NCH * B

    mesh = plsc.VectorSubcoreMesh(core_axis_name="c", subcore_axis_name="s")

    scratch = (
        [pltpu.VMEM((B * POS_PER_W,), jnp.int32)]
        + [pltpu.VMEM((KP, D), jnp.float32) for _ in range(2)]
        + [pltpu.VMEM((KP, D), jnp.float32) for _ in range(NBUF)]
        + [pltpu.SemaphoreType.DMA for _ in range(2 + 2 * NBUF)]
    )

    @functools.partial(
        pl.kernel,
        mesh=mesh,
        out_type=jax.ShapeDtypeStruct((N, D), jnp.float32),
        scratch_types=scratch,
    )
    def body(ids_hbm, wte_hbm, wpe_hbm, out_hbm, idx_v, *bufs):
        pbufs = bufs[0:2]
        gbufs = bufs[2:2 + NBUF]
        psems = bufs[2 + NBUF:4 + NBUF]
        gsems = bufs[4 + NBUF:4 + 2 * NBUF]
        osems = bufs[4 + 2 * NBUF:4 + 3 * NBUF]

        wid = lax.axis_index("s") * NC + lax.axis_index("c")
        p0 = wid * POS_PER_W
        for b in range(B):
            pltpu.sync_copy(ids_hbm.at[pl.ds(b * S + p0, POS_PER_W)],
                            idx_v.at[pl.ds(b * POS_PER_W, POS_PER_W)])

        pcop, gcop, ocop = {}, {}, {}

        def start_pbuf(pc):
            c = pltpu.make_async_copy(
                wpe_hbm.at[pl.ds(p0 + pc * KP, KP)],
                pbufs[pc % 2], psems[pc % 2])
            c.start()
            pcop[pc] = c

        def start_gather(i):
            pc, b = divmod(i, B)
            buf = i % NBUF
            idx = idx_v.at[pl.ds(b * POS_PER_W + pc * KP, KP)]
            c = pltpu.make_async_copy(wte_hbm.at[idx], gbufs[buf], gsems[buf])
            c.start()
            gcop[i] = c

        start_pbuf(0)
        for j in range(NBUF - 1):
            start_gather(j)

        for i in range(NIT):
            pc, b = divmod(i, B)
            buf = i % NBUF
            if b == 0:
                if pc + 1 < NCH:
                    start_pbuf(pc + 1)
                pcop[pc].wait()
            gcop[i].wait()
            pb = pbufs[pc % 2]
            gb = gbufs[buf]
            ipr = D // (_U * _L)

            def addall(jj, _, gb=gb, pb=pb):
                r = jj // ipr
                c0 = (jj % ipr) * (_U * _L)
                for u in range(_U):
                    sl = pl.ds(c0 + u * _L, _L)
                    plsc.addupdate(gb.at[r, sl], pb[r, sl])
                return 0
            lax.fori_loop(0, KP * ipr, addall, 0)
            c = pltpu.make_async_copy(
                gb, out_hbm.at[pl.ds(b * S + p0 + pc * KP, KP)], osems[buf])
            c.start()
            ocop[i] = c
            if i + NBUF - 1 < NIT:
                if i >= 1:
                    ocop[i - 1].wait()
                start_gather(i + NBUF - 1)
        for i in range(NIT - NBUF, NIT):
            ocop[i].wait()

    return body


def kernel(input_ids, attention_mask, wte, wpe):
    B_, S_ = input_ids.shape
    D_ = wte.shape[1]
    ids_flat = input_ids.reshape(-1).astype(jnp.int32)
    fn = _make_embed(D_, B_, S_)
    out = fn(ids_flat, wte, wpe)
    hidden = out.reshape(B_, S_, D_)
    return (hidden, input_ids.reshape(-1, S_), attention_mask)


# speedup vs baseline: 1.2129x; 1.2129x over previous
import functools

import jax
import jax.numpy as jnp
from jax import lax
from jax.experimental import pallas as pl
from jax.experimental.pallas import tpu as pltpu
from jax.experimental.pallas import tpu_sc as plsc

_L = 16
_U = 4


@functools.lru_cache(maxsize=None)
def _make_embed(D, B, S):
    info = plsc.get_sparse_core_info()
    NC, NS = info.num_cores, info.num_subcores
    NW = NC * NS
    N = B * S
    POS_PER_W = S // NW
    KP = 8
    NCH = POS_PER_W // KP
    NBUF = 5
    NIT = NCH * B

    mesh = plsc.VectorSubcoreMesh(core_axis_name="c", subcore_axis_name="s")

    scratch = (
        [pltpu.VMEM((B * POS_PER_W,), jnp.int32)]
        + [pltpu.VMEM((KP, D), jnp.float32) for _ in range(2)]
        + [pltpu.VMEM((KP, D), jnp.float32) for _ in range(NBUF)]
        + [pltpu.SemaphoreType.DMA for _ in range(2 + 2 * NBUF)]
    )

    @functools.partial(
        pl.kernel,
        mesh=mesh,
        out_type=jax.ShapeDtypeStruct((B, S, D), jnp.float32),
        scratch_types=scratch,
    )
    def body(ids_hbm, wte_hbm, wpe_hbm, out_hbm, idx_v, *bufs):
        pbufs = bufs[0:2]
        gbufs = bufs[2:2 + NBUF]
        psems = bufs[2 + NBUF:4 + NBUF]
        gsems = bufs[4 + NBUF:4 + 2 * NBUF]
        osems = bufs[4 + 2 * NBUF:4 + 3 * NBUF]

        wid = lax.axis_index("s") * NC + lax.axis_index("c")
        p0 = wid * POS_PER_W
        for b in range(B):
            pltpu.sync_copy(ids_hbm.at[b, pl.ds(p0, POS_PER_W)],
                            idx_v.at[pl.ds(b * POS_PER_W, POS_PER_W)])

        pcop, gcop, ocop = {}, {}, {}

        def start_pbuf(pc):
            c = pltpu.make_async_copy(
                wpe_hbm.at[pl.ds(p0 + pc * KP, KP)],
                pbufs[pc % 2], psems[pc % 2])
            c.start()
            pcop[pc] = c

        def start_gather(i):
            pc, b = divmod(i, B)
            buf = i % NBUF
            idx = idx_v.at[pl.ds(b * POS_PER_W + pc * KP, KP)]
            c = pltpu.make_async_copy(wte_hbm.at[idx], gbufs[buf], gsems[buf])
            c.start()
            gcop[i] = c

        start_pbuf(0)
        for j in range(NBUF - 1):
            start_gather(j)

        for i in range(NIT):
            pc, b = divmod(i, B)
            buf = i % NBUF
            if b == 0:
                if pc + 1 < NCH:
                    start_pbuf(pc + 1)
                pcop[pc].wait()
            gcop[i].wait()
            pb = pbufs[pc % 2]
            gb = gbufs[buf]
            ipr = D // (_U * _L)

            def addall(jj, _, gb=gb, pb=pb):
                r = jj // ipr
                c0 = (jj % ipr) * (_U * _L)
                for u in range(_U):
                    sl = pl.ds(c0 + u * _L, _L)
                    plsc.addupdate(gb.at[r, sl], pb[r, sl])
                return 0
            lax.fori_loop(0, KP * ipr, addall, 0)
            c = pltpu.make_async_copy(
                gb, out_hbm.at[b, pl.ds(p0 + pc * KP, KP)], osems[buf])
            c.start()
            ocop[i] = c
            if i + NBUF - 1 < NIT:
                if i >= 1:
                    ocop[i - 1].wait()
                start_gather(i + NBUF - 1)
        for i in range(NIT - NBUF, NIT):
            ocop[i].wait()

    return body


def kernel(input_ids, attention_mask, wte, wpe):
    B_, S_ = input_ids.shape
    D_ = wte.shape[1]
    fn = _make_embed(D_, B_, S_)
    hidden = fn(input_ids.astype(jnp.int32), wte, wpe)
    return (hidden, input_ids.reshape(-1, S_), attention_mask)
